# baseline (speedup 1.0000x reference)
import jax
import jax.numpy as jnp
from jax import lax
from jax.experimental import pallas as pl
from jax.experimental.pallas import tpu as pltpu
from jax.experimental.pallas import tpu_sc as plsc

NC, NS, L = 2, 16, 16
NW = NC * NS
E = 320000
C = 80
TOTAL_CHUNKS = E // (NW * C)
SPLIT_CHUNKS = (23, 47, 55)
GROUPS = C // L
D = 128
W = D // 2


def _lane_shuffle(x, idx):
    dn = lax.GatherDimensionNumbers(
        offset_dims=(), collapsed_slice_dims=(0,), start_index_map=(0,))
    return lax.gather(x, idx[:, None], dn, (1,),
                      mode=lax.GatherScatterMode.PROMISE_IN_BOUNDS)


def _body(chunks, emb_hbm, idx_hbm, out_hbm,
          idx_all, rows0, rows1, out_v, sem0, sem1):
    e_per = chunks * C
    wid = lax.axis_index("s") * NC + lax.axis_index("c")
    base_e = wid * e_per
    lane = lax.iota(jnp.int32, L)

    pltpu.sync_copy(idx_hbm.at[pl.ds(2 * base_e, 2 * e_per)], idx_all)

    rows = (rows0, rows1)
    sems = (sem0, sem1)

    def gather_descr(j, b):
        src = emb_hbm.at[idx_all.at[pl.ds(j * 2 * C, 2 * C)]]
        return src, rows[b], sems[b]

    def start_gather(j, b):
        src, dst, sem = gather_descr(j, b)
        pltpu.async_copy(src, dst, sem)

    def wait_gather(j, b):
        src, dst, sem = gather_descr(j, b)
        pltpu.make_async_copy(src, dst, sem).wait()

    def combine(a, b, sh):
        mask = (lane & sh) == 0
        x = jnp.where(mask, a, b)
        y = jnp.where(mask, b, a)
        return x + _lane_shuffle(y, lane ^ sh)

    def compute(j, b):
        rv = rows[b]

        def group(g, carry2):
            accs = []
            for m in range(L):
                k = g * L + m
                fr = rv.at[2 * k]
                to = rv.at[2 * k + 1]
                acc = None
                for c in range(W // L):
                    fw = fr[pl.ds(c * L, L)]
                    tw = to[pl.ds(c * L, L)]
                    fa = lax.bitcast_convert_type(fw << 16, jnp.float32)
                    fb = lax.bitcast_convert_type(fw & jnp.int32(-65536),
                                                  jnp.float32)
                    ta = lax.bitcast_convert_type(tw << 16, jnp.float32)
                    tb = lax.bitcast_convert_type(tw & jnp.int32(-65536),
                                                  jnp.float32)
                    p = fa * ta + fb * tb
                    acc = p if acc is None else acc + p
                accs.append(acc)
            sh = 1
            while len(accs) > 1:
                accs = [combine(accs[2 * i], accs[2 * i + 1], sh)
                        for i in range(len(accs) // 2)]
                sh *= 2
            out_v[pl.ds(g * L, L)] = accs[0]
            return carry2

        lax.fori_loop(0, GROUPS, group, 0)
        pltpu.sync_copy(out_v, out_hbm.at[pl.ds(base_e + j * C, C)])

    def do_chunk(j, b, has_next):
        if has_next:
            start_gather(j + 1, 1 - b)
        wait_gather(j, b)
        compute(j, b)

    start_gather(0, 0)

    def pair(p, carry):
        j = 2 * p
        do_chunk(j, 0, True)
        do_chunk(j + 1, 1, True)
        return carry

    if chunks % 2 == 1:
        lax.fori_loop(0, (chunks - 1) // 2, pair, 0)
        do_chunk(chunks - 1, 0, False)
    else:
        lax.fori_loop(0, (chunks - 2) // 2, pair, 0)
        do_chunk(chunks - 2, 0, True)
        do_chunk(chunks - 1, 1, False)


def kernel(embedding, indices):
    emb_bf = embedding.astype(jnp.bfloat16)
    emb_i32 = lax.bitcast_convert_type(
        emb_bf.reshape(emb_bf.shape[0], W, 2), jnp.int32)
    mesh = plsc.VectorSubcoreMesh(core_axis_name="c", subcore_axis_name="s")
    idx_pairs = indices.astype(jnp.int32)
    outs = []
    e0 = 0
    for chunks in SPLIT_CHUNKS:
        es = chunks * C * NW
        f = pl.kernel(
            lambda *args, _c=chunks: _body(_c, *args),
            out_type=jax.ShapeDtypeStruct((es,), jnp.float32),
            mesh=mesh,
            compiler_params=pltpu.CompilerParams(use_tc_tiling_on_sc=False),
            scratch_types=[
                pltpu.VMEM((2 * chunks * C,), jnp.int32),
                pltpu.VMEM((2 * C, W), jnp.int32),
                pltpu.VMEM((2 * C, W), jnp.int32),
                pltpu.VMEM((C,), jnp.float32),
                pltpu.SemaphoreType.DMA,
                pltpu.SemaphoreType.DMA,
            ],
        )
        idx_flat_s = idx_pairs[e0:e0 + es, :].reshape(-1)
        outs.append(f(emb_i32, idx_flat_s))
        e0 += es
    return jnp.concatenate(outs)


# speedup vs baseline: 1.0084x; 1.0084x over previous
import jax
import jax.numpy as jnp
from jax import lax
from jax.experimental import pallas as pl
from jax.experimental.pallas import tpu as pltpu
from jax.experimental.pallas import tpu_sc as plsc

NC, NS, L = 2, 16, 16
NW = NC * NS
E = 320000
C = 80
TOTAL_CHUNKS = E // (NW * C)
SPLIT_CHUNKS = (31, 47, 47)
GROUPS = C // L
D = 128
W = D // 2


def _lane_shuffle(x, idx):
    dn = lax.GatherDimensionNumbers(
        offset_dims=(), collapsed_slice_dims=(0,), start_index_map=(0,))
    return lax.gather(x, idx[:, None], dn, (1,),
                      mode=lax.GatherScatterMode.PROMISE_IN_BOUNDS)


def _body(chunks, emb_hbm, idx_hbm, out_hbm,
          idx_all, rows0, rows1, out_v, sem0, sem1):
    e_per = chunks * C
    wid = lax.axis_index("s") * NC + lax.axis_index("c")
    base_e = wid * e_per
    lane = lax.iota(jnp.int32, L)

    pltpu.sync_copy(idx_hbm.at[pl.ds(2 * base_e, 2 * e_per)], idx_all)

    rows = (rows0, rows1)
    sems = (sem0, sem1)

    def gather_descr(j, b):
        src = emb_hbm.at[idx_all.at[pl.ds(j * 2 * C, 2 * C)]]
        return src, rows[b], sems[b]

    def start_gather(j, b):
        src, dst, sem = gather_descr(j, b)
        pltpu.async_copy(src, dst, sem)

    def wait_gather(j, b):
        src, dst, sem = gather_descr(j, b)
        pltpu.make_async_copy(src, dst, sem).wait()

    def combine(a, b, sh):
        mask = (lane & sh) == 0
        x = jnp.where(mask, a, b)
        y = jnp.where(mask, b, a)
        return x + _lane_shuffle(y, lane ^ sh)

    def compute(j, b):
        rv = rows[b]

        def group(g, carry2):
            accs = []
            for m in range(L):
                k = g * L + m
                fr = rv.at[2 * k]
                to = rv.at[2 * k + 1]
                acc = None
                for c in range(W // L):
                    fw = fr[pl.ds(c * L, L)]
                    tw = to[pl.ds(c * L, L)]
                    fa = lax.bitcast_convert_type(fw << 16, jnp.float32)
                    fb = lax.bitcast_convert_type(fw & jnp.int32(-65536),
                                                  jnp.float32)
                    ta = lax.bitcast_convert_type(tw << 16, jnp.float32)
                    tb = lax.bitcast_convert_type(tw & jnp.int32(-65536),
                                                  jnp.float32)
                    p = fa * ta + fb * tb
                    acc = p if acc is None else acc + p
                accs.append(acc)
            sh = 1
            while len(accs) > 1:
                accs = [combine(accs[2 * i], accs[2 * i + 1], sh)
                        for i in range(len(accs) // 2)]
                sh *= 2
            out_v[pl.ds(g * L, L)] = accs[0]
            return carry2

        lax.fori_loop(0, GROUPS, group, 0)
        pltpu.sync_copy(out_v, out_hbm.at[pl.ds(base_e + j * C, C)])

    def do_chunk(j, b, has_next):
        if has_next:
            start_gather(j + 1, 1 - b)
        wait_gather(j, b)
        compute(j, b)

    start_gather(0, 0)

    def pair(p, carry):
        j = 2 * p
        do_chunk(j, 0, True)
        do_chunk(j + 1, 1, True)
        return carry

    if chunks % 2 == 1:
        lax.fori_loop(0, (chunks - 1) // 2, pair, 0)
        do_chunk(chunks - 1, 0, False)
    else:
        lax.fori_loop(0, (chunks - 2) // 2, pair, 0)
        do_chunk(chunks - 2, 0, True)
        do_chunk(chunks - 1, 1, False)


def kernel(embedding, indices):
    emb_bf = embedding.astype(jnp.bfloat16)
    emb_i32 = lax.bitcast_convert_type(
        emb_bf.reshape(emb_bf.shape[0], W, 2), jnp.int32)
    mesh = plsc.VectorSubcoreMesh(core_axis_name="c", subcore_axis_name="s")
    idx_pairs = indices.astype(jnp.int32)
    outs = []
    e0 = 0
    for chunks in SPLIT_CHUNKS:
        es = chunks * C * NW
        f = pl.kernel(
            lambda *args, _c=chunks: _body(_c, *args),
            out_type=jax.ShapeDtypeStruct((es,), jnp.float32),
            mesh=mesh,
            compiler_params=pltpu.CompilerParams(use_tc_tiling_on_sc=False),
            scratch_types=[
                pltpu.VMEM((2 * chunks * C,), jnp.int32),
                pltpu.VMEM((2 * C, W), jnp.int32),
                pltpu.VMEM((2 * C, W), jnp.int32),
                pltpu.VMEM((C,), jnp.float32),
                pltpu.SemaphoreType.DMA,
                pltpu.SemaphoreType.DMA,
            ],
        )
        idx_flat_s = idx_pairs[e0:e0 + es, :].reshape(-1)
        outs.append(f(emb_i32, idx_flat_s))
        e0 += es
    return jnp.concatenate(outs)
